# baseline (speedup 1.0000x reference)
import jax
import jax.numpy as jnp
from jax import lax
from jax.experimental import pallas as pl
from jax.experimental.pallas import tpu as pltpu
from jax.experimental.pallas import tpu_sc as plsc

_NC = 2
_NS = 16
_NW = _NC * _NS
_L = 16

_CHUNK = 128
_NBUF = 4


def _make_lookup(batch, fields, feat, num_emb):
    assert batch % _NW == 0
    b_per_w = batch // _NW
    nq = b_per_w // _CHUNK
    assert nq == _NBUF and fields % 2 == 0
    nu = fields * nq

    mesh = plsc.VectorSubcoreMesh(
        core_axis_name="c", subcore_axis_name="s",
        num_cores=_NC, num_subcores=_NS)

    @pl.kernel(
        out_type=jax.ShapeDtypeStruct((fields, feat, batch), jnp.float32),
        mesh=mesh,
        compiler_params=pltpu.CompilerParams(
            use_tc_tiling_on_sc=False, needs_layout_passes=False),
        scratch_types=[
            pltpu.VMEM((fields * b_per_w,), jnp.int32),
            pltpu.VMEM((fields * b_per_w,), jnp.int32),
            pltpu.VMEM((_CHUNK, 128), jnp.float32),
            pltpu.VMEM((_CHUNK, 128), jnp.float32),
            pltpu.VMEM((_CHUNK, 128), jnp.float32),
            pltpu.VMEM((_CHUNK, 128), jnp.float32),
            pltpu.VMEM((feat, b_per_w), jnp.float32),
            pltpu.VMEM((feat, b_per_w), jnp.float32),
            pltpu.SemaphoreType.DMA,
            pltpu.SemaphoreType.DMA,
            pltpu.SemaphoreType.DMA,
            pltpu.SemaphoreType.DMA,
            pltpu.SemaphoreType.DMA,
            pltpu.SemaphoreType.DMA,
            pltpu.SemaphoreType.DMA,
        ],
    )
    def k(table_hbm, idx_hbm, out_hbm, idx_all, pidx_all, g0, g1, g2, g3,
          ot0, ot1, isem, gsem0, gsem1, gsem2, gsem3, osem0, osem1):
        wid = lax.axis_index("s") * _NC + lax.axis_index("c")
        b0 = wid * b_per_w
        iota = lax.iota(jnp.int32, _L)

        gbuf = (g0, g1, g2, g3)
        gsems = (gsem0, gsem1, gsem2, gsem3)
        otile = (ot0, ot1)
        osems = (osem0, osem1)

        icps = [
            pltpu.async_copy(
                idx_hbm.at[pl.ds(f * batch + b0, b_per_w)],
                idx_all.at[pl.ds(f * b_per_w, b_per_w)], isem)
            for f in range(fields)
        ]
        for cp in icps:
            cp.wait()

        def pidx_body(t, carry):
            v = idx_all[pl.ds(t * _L, _L)]
            pidx_all[pl.ds(t * _L, _L)] = lax.shift_right_logical(v, 3)
            return carry
        lax.fori_loop(0, fields * b_per_w // _L, pidx_body, 0)

        def ichunk(u):
            return pidx_all.at[pl.ds(u * _CHUNK, _CHUNK)]

        def fire(u, par):
            pltpu.async_copy(table_hbm.at[ichunk(u)], gbuf[par], gsems[par])

        def gwait(u, par):
            pltpu.make_async_copy(
                table_hbm.at[ichunk(u)], gbuf[par], gsems[par]).wait()

        def extract(u, par, o_ref, col0):
            g = gbuf[par]
            for t in range(_CHUNK // _L):
                rows = iota + t * _L
                v = idx_all[pl.ds(u * _CHUNK + t * _L, _L)]
                off = lax.shift_left(jnp.bitwise_and(v, 7), 4)
                for j in range(feat):
                    o_ref[j, pl.ds(col0 + t * _L, _L)] = (
                        plsc.load_gather(g, [rows, off + j]))

        def oslice(f):
            return out_hbm.at[f, :, pl.ds(b0, b_per_w)]

        for u in range(_NBUF - 1):
            fire(u, u)

        def body(s, carry):
            u0 = s * 2 * nq
            fa = s * 2
            fb = s * 2 + 1

            @pl.when(s > 0)
            def _():
                pltpu.make_async_copy(otile[0], oslice(fa), osems[0]).wait()
                pltpu.make_async_copy(otile[1], oslice(fb), osems[1]).wait()

            for p in range(2 * nq):
                u = u0 + p
                par = p % _NBUF

                @pl.when(u + _NBUF - 1 < nu)
                def _():
                    fire(u + _NBUF - 1, (p + _NBUF - 1) % _NBUF)
                gwait(u, par)
                extract(u, par, otile[p // nq], (p % nq) * _CHUNK)

            pltpu.async_copy(otile[0], oslice(fa), osems[0])
            pltpu.async_copy(otile[1], oslice(fb), osems[1])
            return carry

        lax.fori_loop(0, fields // 2, body, 0)
        pltpu.make_async_copy(otile[0], oslice(fields - 2), osems[0]).wait()
        pltpu.make_async_copy(otile[1], oslice(fields - 1), osems[1]).wait()

    return k


def kernel(inputs, embedding):
    batch, fields = inputs.shape
    num_emb, feat = embedding.shape
    idx_fm = jnp.transpose(inputs).reshape(batch * fields).astype(jnp.int32)
    packed = jnp.reshape(embedding, (num_emb // 8, 128))
    call = _make_lookup(batch, fields, feat, num_emb)
    out_t = call(packed, idx_fm)
    return jnp.transpose(out_t, (2, 0, 1))


# speedup vs baseline: 1.0744x; 1.0744x over previous
import jax
import jax.numpy as jnp
from jax import lax
from jax.experimental import pallas as pl
from jax.experimental.pallas import tpu as pltpu
from jax.experimental.pallas import tpu_sc as plsc

_NC = 2
_NS = 16
_NW = _NC * _NS
_L = 16

_CHUNK = 128
_NBUF = 4


def _make_relayout(num_emb, feat):
    n_packed = num_emb // 8
    n_full = num_emb // 128
    tail = num_emb - n_full * 128
    per_w = n_full // _NW
    rem = n_full - per_w * _NW
    nslots = (per_w + (1 if rem else 0) + _NBUF - 1) // _NBUF

    mesh = plsc.VectorSubcoreMesh(
        core_axis_name="c", subcore_axis_name="s",
        num_cores=_NC, num_subcores=_NS)

    @pl.kernel(
        out_type=jax.ShapeDtypeStruct((n_packed, 128), jnp.float32),
        mesh=mesh,
        compiler_params=pltpu.CompilerParams(
            use_tc_tiling_on_sc=True, needs_layout_passes=False,
            disable_bounds_checks=True),
        scratch_types=[
            pltpu.VMEM((feat, 128), jnp.float32),
            pltpu.VMEM((feat, 128), jnp.float32),
            pltpu.VMEM((feat, 128), jnp.float32),
            pltpu.VMEM((feat, 128), jnp.float32),
            pltpu.VMEM((feat, 128), jnp.float32),
            pltpu.VMEM((feat, 128), jnp.float32),
            pltpu.SemaphoreType.DMA,
            pltpu.SemaphoreType.DMA,
            pltpu.SemaphoreType.DMA,
            pltpu.SemaphoreType.DMA,
            pltpu.SemaphoreType.DMA,
            pltpu.SemaphoreType.DMA,
        ],
    )
    def k(emb_t_hbm, tailp_hbm, packed_hbm, a0, a1, a2, a3, o0, o1,
          s0, s1, s2, s3, os0, os1):
        wid = lax.axis_index("s") * _NC + lax.axis_index("c")
        iota = lax.iota(jnp.int32, _L)
        abuf = (a0, a1, a2, a3)
        asem = (s0, s1, s2, s3)
        obuf = (o0, o1)
        osem = (os0, os1)

        def cof(t):
            return wid + t * _NW

        def valid(t):
            return cof(t) < n_full

        def fire(t, par):
            pltpu.async_copy(
                emb_t_hbm.at[:, pl.ds(pl.multiple_of(cof(t) * 128, 128), 128)],
                abuf[par], asem[par])

        def await_in(par):
            pltpu.make_async_copy(
                emb_t_hbm.at[:, pl.ds(0, 128)], abuf[par], asem[par]).wait()

        def transpose(src, dst, hi=8):
            for aa in range(hi):
                for g in range(_L):
                    l = aa * _L + g
                    vals = plsc.load_gather(
                        src, [iota, jnp.full((_L,), l, jnp.int32)])
                    dst[aa * 2 + g // 8, pl.ds((g % 8) * _L, _L)] = vals

        def flush(t, opar):
            pltpu.async_copy(
                obuf[opar],
                packed_hbm.at[pl.ds(pl.multiple_of(cof(t) * 16, 16), 16), :],
                osem[opar])

        def drain_out(opar):
            pltpu.make_async_copy(
                obuf[opar], packed_hbm.at[pl.ds(0, 16), :],
                osem[opar]).wait()

        for t in range(_NBUF - 1):
            @pl.when(valid(t))
            def _(t=t):
                fire(t, t)

        def body(s, carry):
            for p in range(_NBUF):
                t = s * _NBUF + p

                @pl.when(valid(t + _NBUF - 1))
                def _():
                    fire(t + _NBUF - 1, (p + _NBUF - 1) % _NBUF)

                @pl.when(valid(t))
                def _():
                    await_in(p)

                    @pl.when(t >= 2)
                    def _():
                        drain_out(p % 2)
                    transpose(abuf[p], obuf[p % 2])
                    flush(t, p % 2)
            return carry

        lax.fori_loop(0, nslots, body, 0)

        drain_out(0)
        drain_out(1)

        if tail:
            @pl.when(wid == 0)
            def _():
                pltpu.sync_copy(tailp_hbm, obuf[0].at[pl.ds(0, tail // 8), :])
                pltpu.sync_copy(
                    obuf[0].at[pl.ds(0, tail // 8), :],
                    packed_hbm.at[pl.ds(n_full * 16, tail // 8), :])

    return k


def _make_lookup(batch, fields, feat, num_emb):
    assert batch % _NW == 0
    b_per_w = batch // _NW
    nq = b_per_w // _CHUNK
    assert nq == _NBUF and fields % 2 == 0
    nu = fields * nq

    mesh = plsc.VectorSubcoreMesh(
        core_axis_name="c", subcore_axis_name="s",
        num_cores=_NC, num_subcores=_NS)

    @pl.kernel(
        out_type=jax.ShapeDtypeStruct((fields, feat, batch), jnp.float32),
        mesh=mesh,
        compiler_params=pltpu.CompilerParams(
            use_tc_tiling_on_sc=False, needs_layout_passes=False),
        scratch_types=[
            pltpu.VMEM((fields * b_per_w,), jnp.int32),
            pltpu.VMEM((fields * b_per_w,), jnp.int32),
            pltpu.VMEM((_CHUNK, 128), jnp.float32),
            pltpu.VMEM((_CHUNK, 128), jnp.float32),
            pltpu.VMEM((_CHUNK, 128), jnp.float32),
            pltpu.VMEM((_CHUNK, 128), jnp.float32),
            pltpu.VMEM((feat, b_per_w), jnp.float32),
            pltpu.VMEM((feat, b_per_w), jnp.float32),
            pltpu.SemaphoreType.DMA,
            pltpu.SemaphoreType.DMA,
            pltpu.SemaphoreType.DMA,
            pltpu.SemaphoreType.DMA,
            pltpu.SemaphoreType.DMA,
            pltpu.SemaphoreType.DMA,
            pltpu.SemaphoreType.DMA,
        ],
    )
    def k(table_hbm, idx_hbm, out_hbm, idx_all, pidx_all, g0, g1, g2, g3,
          ot0, ot1, isem, gsem0, gsem1, gsem2, gsem3, osem0, osem1):
        wid = lax.axis_index("s") * _NC + lax.axis_index("c")
        b0 = wid * b_per_w
        iota = lax.iota(jnp.int32, _L)

        gbuf = (g0, g1, g2, g3)
        gsems = (gsem0, gsem1, gsem2, gsem3)
        otile = (ot0, ot1)
        osems = (osem0, osem1)

        icps = [
            pltpu.async_copy(
                idx_hbm.at[pl.ds(f * batch + b0, b_per_w)],
                idx_all.at[pl.ds(f * b_per_w, b_per_w)], isem)
            for f in range(fields)
        ]
        for cp in icps:
            cp.wait()

        def pidx_body(t, carry):
            v = idx_all[pl.ds(t * _L, _L)]
            pidx_all[pl.ds(t * _L, _L)] = lax.shift_right_logical(v, 3)
            return carry
        lax.fori_loop(0, fields * b_per_w // _L, pidx_body, 0)

        def ichunk(u):
            return pidx_all.at[pl.ds(u * _CHUNK, _CHUNK)]

        def fire(u, par):
            pltpu.async_copy(table_hbm.at[ichunk(u)], gbuf[par], gsems[par])

        def gwait(u, par):
            pltpu.make_async_copy(
                table_hbm.at[ichunk(u)], gbuf[par], gsems[par]).wait()

        def extract(u, par, o_ref, col0):
            g = gbuf[par]
            for t in range(_CHUNK // _L):
                rows = iota + t * _L
                v = idx_all[pl.ds(u * _CHUNK + t * _L, _L)]
                off = lax.shift_left(jnp.bitwise_and(v, 7), 4)
                for j in range(feat):
                    o_ref[j, pl.ds(col0 + t * _L, _L)] = (
                        plsc.load_gather(g, [rows, off + j]))

        def oslice(f):
            return out_hbm.at[f, :, pl.ds(b0, b_per_w)]

        for u in range(_NBUF - 1):
            fire(u, u)

        def body(s, carry):
            u0 = s * 2 * nq
            fa = s * 2
            fb = s * 2 + 1

            @pl.when(s > 0)
            def _():
                pltpu.make_async_copy(otile[0], oslice(fa), osems[0]).wait()
                pltpu.make_async_copy(otile[1], oslice(fb), osems[1]).wait()

            for p in range(2 * nq):
                u = u0 + p
                par = p % _NBUF

                @pl.when(u + _NBUF - 1 < nu)
                def _():
                    fire(u + _NBUF - 1, (p + _NBUF - 1) % _NBUF)
                gwait(u, par)
                extract(u, par, otile[p // nq], (p % nq) * _CHUNK)

            pltpu.async_copy(otile[0], oslice(fa), osems[0])
            pltpu.async_copy(otile[1], oslice(fb), osems[1])
            return carry

        lax.fori_loop(0, fields // 2, body, 0)
        pltpu.make_async_copy(otile[0], oslice(fields - 2), osems[0]).wait()
        pltpu.make_async_copy(otile[1], oslice(fields - 1), osems[1]).wait()

    return k


def kernel(inputs, embedding):
    batch, fields = inputs.shape
    num_emb, feat = embedding.shape
    idx_fm = jnp.transpose(inputs).reshape(batch * fields).astype(jnp.int32)
    emb_t = jnp.transpose(embedding)
    n_tail = num_emb - (num_emb // 128) * 128
    tail_packed = jnp.reshape(
        embedding[num_emb - n_tail:], (n_tail // 8, 8 * feat))
    packed = _make_relayout(num_emb, feat)(emb_t, tail_packed)
    call = _make_lookup(batch, fields, feat, num_emb)
    out_t = call(packed, idx_fm)
    return jnp.transpose(out_t, (2, 0, 1))


# speedup vs baseline: 1.4745x; 1.3724x over previous
import jax
import jax.numpy as jnp
from jax import lax
from jax.experimental import pallas as pl
from jax.experimental.pallas import tpu as pltpu
from jax.experimental.pallas import tpu_sc as plsc

_NC = 2
_NS = 16
_NW = _NC * _NS
_L = 16

_CHUNK = 128
_NBUF = 4


def _make_relayout(num_emb, feat):
    n_packed = num_emb // 8
    n_full = num_emb // 128
    tail = num_emb - n_full * 128
    per_w = n_full // _NW
    rem = n_full - per_w * _NW
    nslots = (per_w + (1 if rem else 0) + _NBUF - 1) // _NBUF

    mesh = plsc.VectorSubcoreMesh(
        core_axis_name="c", subcore_axis_name="s",
        num_cores=_NC, num_subcores=_NS)

    @pl.kernel(
        out_type=jax.ShapeDtypeStruct((n_packed, 128), jnp.float32),
        mesh=mesh,
        compiler_params=pltpu.CompilerParams(
            use_tc_tiling_on_sc=True, needs_layout_passes=False,
            disable_bounds_checks=True),
        scratch_types=[
            pltpu.VMEM((feat, 128), jnp.float32),
            pltpu.VMEM((feat, 128), jnp.float32),
            pltpu.VMEM((feat, 128), jnp.float32),
            pltpu.VMEM((feat, 128), jnp.float32),
            pltpu.VMEM((feat, 128), jnp.float32),
            pltpu.VMEM((feat, 128), jnp.float32),
            pltpu.SemaphoreType.DMA,
            pltpu.SemaphoreType.DMA,
            pltpu.SemaphoreType.DMA,
            pltpu.SemaphoreType.DMA,
            pltpu.SemaphoreType.DMA,
            pltpu.SemaphoreType.DMA,
        ],
    )
    def k(emb_t_hbm, tailp_hbm, packed_hbm, a0, a1, a2, a3, o0, o1,
          s0, s1, s2, s3, os0, os1):
        wid = lax.axis_index("s") * _NC + lax.axis_index("c")
        iota = lax.iota(jnp.int32, _L)
        abuf = (a0, a1, a2, a3)
        asem = (s0, s1, s2, s3)
        obuf = (o0, o1)
        osem = (os0, os1)

        def cof(t):
            return wid + t * _NW

        def valid(t):
            return cof(t) < n_full

        def fire(t, par):
            pltpu.async_copy(
                emb_t_hbm.at[:, pl.ds(pl.multiple_of(cof(t) * 128, 128), 128)],
                abuf[par], asem[par])

        def await_in(par):
            pltpu.make_async_copy(
                emb_t_hbm.at[:, pl.ds(0, 128)], abuf[par], asem[par]).wait()

        def transpose(src, dst):
            for aa in range(8):
                vals = [
                    plsc.load_gather(
                        src, [iota, jnp.full((_L,), aa * _L + g, jnp.int32)])
                    for g in range(_L)
                ]
                for g in range(_L):
                    dst[aa * 2 + g // 8, pl.ds((g % 8) * _L, _L)] = vals[g]

        def flush(t, opar):
            pltpu.async_copy(
                obuf[opar],
                packed_hbm.at[pl.ds(pl.multiple_of(cof(t) * 16, 16), 16), :],
                osem[opar])

        def drain_out(opar):
            pltpu.make_async_copy(
                obuf[opar], packed_hbm.at[pl.ds(0, 16), :],
                osem[opar]).wait()

        for t in range(_NBUF - 1):
            @pl.when(valid(t))
            def _(t=t):
                fire(t, t)

        def body(s, carry):
            for p in range(_NBUF):
                t = s * _NBUF + p

                @pl.when(valid(t + _NBUF - 1))
                def _():
                    fire(t + _NBUF - 1, (p + _NBUF - 1) % _NBUF)

                @pl.when(valid(t))
                def _():
                    await_in(p)

                    @pl.when(t >= 2)
                    def _():
                        drain_out(p % 2)
                    transpose(abuf[p], obuf[p % 2])
                    flush(t, p % 2)
            return carry

        lax.fori_loop(0, nslots, body, 0)

        drain_out(0)
        drain_out(1)

        if tail:
            @pl.when(wid == 0)
            def _():
                pltpu.sync_copy(tailp_hbm, obuf[0].at[pl.ds(0, tail // 8), :])
                pltpu.sync_copy(
                    obuf[0].at[pl.ds(0, tail // 8), :],
                    packed_hbm.at[pl.ds(n_full * 16, tail // 8), :])

    return k


def _make_lookup(batch, fields, feat, num_emb):
    assert batch % _NW == 0
    b_per_w = batch // _NW
    nq = b_per_w // _CHUNK
    assert nq == _NBUF and fields % 2 == 0
    nu = fields * nq

    mesh = plsc.VectorSubcoreMesh(
        core_axis_name="c", subcore_axis_name="s",
        num_cores=_NC, num_subcores=_NS)

    @pl.kernel(
        out_type=jax.ShapeDtypeStruct((fields, feat, batch), jnp.float32),
        mesh=mesh,
        compiler_params=pltpu.CompilerParams(
            use_tc_tiling_on_sc=False, needs_layout_passes=False),
        scratch_types=[
            pltpu.VMEM((fields * b_per_w,), jnp.int32),
            pltpu.VMEM((fields * b_per_w,), jnp.int32),
            pltpu.VMEM((_CHUNK, 128), jnp.float32),
            pltpu.VMEM((_CHUNK, 128), jnp.float32),
            pltpu.VMEM((_CHUNK, 128), jnp.float32),
            pltpu.VMEM((_CHUNK, 128), jnp.float32),
            pltpu.VMEM((feat, b_per_w), jnp.float32),
            pltpu.VMEM((feat, b_per_w), jnp.float32),
            pltpu.SemaphoreType.DMA,
            pltpu.SemaphoreType.DMA,
            pltpu.SemaphoreType.DMA,
            pltpu.SemaphoreType.DMA,
            pltpu.SemaphoreType.DMA,
            pltpu.SemaphoreType.DMA,
            pltpu.SemaphoreType.DMA,
        ],
    )
    def k(table_hbm, idx_hbm, out_hbm, idx_all, pidx_all, g0, g1, g2, g3,
          ot0, ot1, isem, gsem0, gsem1, gsem2, gsem3, osem0, osem1):
        wid = lax.axis_index("s") * _NC + lax.axis_index("c")
        b0 = wid * b_per_w
        iota = lax.iota(jnp.int32, _L)

        gbuf = (g0, g1, g2, g3)
        gsems = (gsem0, gsem1, gsem2, gsem3)
        otile = (ot0, ot1)
        osems = (osem0, osem1)

        icps = [
            pltpu.async_copy(
                idx_hbm.at[pl.ds(f * batch + b0, b_per_w)],
                idx_all.at[pl.ds(f * b_per_w, b_per_w)], isem)
            for f in range(fields)
        ]
        for cp in icps:
            cp.wait()

        def pidx_body(t, carry):
            v = idx_all[pl.ds(t * _L, _L)]
            pidx_all[pl.ds(t * _L, _L)] = lax.shift_right_logical(v, 3)
            return carry
        lax.fori_loop(0, fields * b_per_w // _L, pidx_body, 0)

        def ichunk(u):
            return pidx_all.at[pl.ds(u * _CHUNK, _CHUNK)]

        def fire(u, par):
            pltpu.async_copy(table_hbm.at[ichunk(u)], gbuf[par], gsems[par])

        def gwait(u, par):
            pltpu.make_async_copy(
                table_hbm.at[ichunk(u)], gbuf[par], gsems[par]).wait()

        def extract(u, par, o_ref, col0):
            g = gbuf[par]
            for t in range(_CHUNK // _L):
                rows = iota + t * _L
                v = idx_all[pl.ds(u * _CHUNK + t * _L, _L)]
                off = lax.shift_left(jnp.bitwise_and(v, 7), 4)
                for j in range(feat):
                    o_ref[j, pl.ds(col0 + t * _L, _L)] = (
                        plsc.load_gather(g, [rows, off + j]))

        def oslice(f):
            return out_hbm.at[f, :, pl.ds(b0, b_per_w)]

        for u in range(_NBUF - 1):
            fire(u, u)

        def body(s, carry):
            u0 = s * 2 * nq
            fa = s * 2
            fb = s * 2 + 1

            @pl.when(s > 0)
            def _():
                pltpu.make_async_copy(otile[0], oslice(fa), osems[0]).wait()
                pltpu.make_async_copy(otile[1], oslice(fb), osems[1]).wait()

            for p in range(2 * nq):
                u = u0 + p
                par = p % _NBUF

                @pl.when(u + _NBUF - 1 < nu)
                def _():
                    fire(u + _NBUF - 1, (p + _NBUF - 1) % _NBUF)
                gwait(u, par)
                extract(u, par, otile[p // nq], (p % nq) * _CHUNK)

            pltpu.async_copy(otile[0], oslice(fa), osems[0])
            pltpu.async_copy(otile[1], oslice(fb), osems[1])
            return carry

        lax.fori_loop(0, fields // 2, body, 0)
        pltpu.make_async_copy(otile[0], oslice(fields - 2), osems[0]).wait()
        pltpu.make_async_copy(otile[1], oslice(fields - 1), osems[1]).wait()

    return k


def kernel(inputs, embedding):
    batch, fields = inputs.shape
    num_emb, feat = embedding.shape
    idx_fm = jnp.transpose(inputs).reshape(batch * fields).astype(jnp.int32)
    emb_t = jnp.transpose(embedding)
    n_tail = num_emb - (num_emb // 128) * 128
    tail_packed = jnp.reshape(
        embedding[num_emb - n_tail:], (n_tail // 8, 8 * feat))
    packed = _make_relayout(num_emb, feat)(emb_t, tail_packed)
    call = _make_lookup(batch, fields, feat, num_emb)
    out_t = call(packed, idx_fm)
    return jnp.transpose(out_t, (2, 0, 1))


# speedup vs baseline: 1.6738x; 1.1352x over previous
import jax
import jax.numpy as jnp
from jax import lax
from jax.experimental import pallas as pl
from jax.experimental.pallas import tpu as pltpu
from jax.experimental.pallas import tpu_sc as plsc

_NC = 2
_NS = 16
_NW = _NC * _NS
_L = 16

_CHUNK = 128
_NBUF = 4


def _make_relayout(num_emb, feat):
    n_packed = num_emb // 8
    n_full = num_emb // 128
    tail = num_emb - n_full * 128
    per_w = n_full // _NW
    rem = n_full - per_w * _NW
    nslots = (per_w + (1 if rem else 0) + _NBUF - 1) // _NBUF

    mesh = plsc.VectorSubcoreMesh(
        core_axis_name="c", subcore_axis_name="s",
        num_cores=_NC, num_subcores=_NS)

    @pl.kernel(
        out_type=jax.ShapeDtypeStruct((n_packed, 128), jnp.float32),
        mesh=mesh,
        compiler_params=pltpu.CompilerParams(
            use_tc_tiling_on_sc=True, needs_layout_passes=False,
            disable_bounds_checks=True),
        scratch_types=[
            pltpu.VMEM((feat, 128), jnp.float32),
            pltpu.VMEM((feat, 128), jnp.float32),
            pltpu.VMEM((feat, 128), jnp.float32),
            pltpu.VMEM((feat, 128), jnp.float32),
            pltpu.VMEM((feat, 128), jnp.float32),
            pltpu.VMEM((feat, 128), jnp.float32),
            pltpu.SemaphoreType.DMA,
            pltpu.SemaphoreType.DMA,
            pltpu.SemaphoreType.DMA,
            pltpu.SemaphoreType.DMA,
            pltpu.SemaphoreType.DMA,
            pltpu.SemaphoreType.DMA,
        ],
    )
    def k(emb_t_hbm, tailp_hbm, packed_hbm, a0, a1, a2, a3, o0, o1,
          s0, s1, s2, s3, os0, os1):
        wid = lax.axis_index("s") * _NC + lax.axis_index("c")
        iota = lax.iota(jnp.int32, _L)
        abuf = (a0, a1, a2, a3)
        asem = (s0, s1, s2, s3)
        obuf = (o0, o1)
        osem = (os0, os1)

        def cof(t):
            return wid + t * _NW

        def valid(t):
            return cof(t) < n_full

        def fire(t, par):
            pltpu.async_copy(
                emb_t_hbm.at[:, pl.ds(pl.multiple_of(cof(t) * 128, 128), 128)],
                abuf[par], asem[par])

        def await_in(par):
            pltpu.make_async_copy(
                emb_t_hbm.at[:, pl.ds(0, 128)], abuf[par], asem[par]).wait()

        def transpose(src, dst):
            for ab in range(4):
                vals = [
                    plsc.load_gather(
                        src, [iota, jnp.full((_L,), ab * 32 + g, jnp.int32)])
                    for g in range(32)
                ]
                for g in range(32):
                    l = ab * 32 + g
                    dst[l // 8, pl.ds((l % 8) * _L, _L)] = vals[g]

        def flush(t, opar):
            pltpu.async_copy(
                obuf[opar],
                packed_hbm.at[pl.ds(pl.multiple_of(cof(t) * 16, 16), 16), :],
                osem[opar])

        def drain_out(opar):
            pltpu.make_async_copy(
                obuf[opar], packed_hbm.at[pl.ds(0, 16), :],
                osem[opar]).wait()

        for t in range(_NBUF - 1):
            @pl.when(valid(t))
            def _(t=t):
                fire(t, t)

        def body(s, carry):
            for p in range(_NBUF):
                t = s * _NBUF + p

                @pl.when(valid(t + _NBUF - 1))
                def _():
                    fire(t + _NBUF - 1, (p + _NBUF - 1) % _NBUF)

                @pl.when(valid(t))
                def _():
                    await_in(p)

                    @pl.when(t >= 2)
                    def _():
                        drain_out(p % 2)
                    transpose(abuf[p], obuf[p % 2])
                    flush(t, p % 2)
            return carry

        lax.fori_loop(0, nslots, body, 0)

        drain_out(0)
        drain_out(1)

        if tail:
            @pl.when(wid == 0)
            def _():
                pltpu.sync_copy(tailp_hbm, obuf[0].at[pl.ds(0, tail // 8), :])
                pltpu.sync_copy(
                    obuf[0].at[pl.ds(0, tail // 8), :],
                    packed_hbm.at[pl.ds(n_full * 16, tail // 8), :])

    return k


def _make_lookup(batch, fields, feat, num_emb):
    assert batch % _NW == 0
    b_per_w = batch // _NW
    nq = b_per_w // _CHUNK
    assert nq == _NBUF and fields % 2 == 0
    nu = fields * nq

    mesh = plsc.VectorSubcoreMesh(
        core_axis_name="c", subcore_axis_name="s",
        num_cores=_NC, num_subcores=_NS)

    @pl.kernel(
        out_type=jax.ShapeDtypeStruct((fields, feat, batch), jnp.float32),
        mesh=mesh,
        compiler_params=pltpu.CompilerParams(
            use_tc_tiling_on_sc=False, needs_layout_passes=False),
        scratch_types=[
            pltpu.VMEM((fields * b_per_w,), jnp.int32),
            pltpu.VMEM((fields * b_per_w,), jnp.int32),
            pltpu.VMEM((_CHUNK, 128), jnp.float32),
            pltpu.VMEM((_CHUNK, 128), jnp.float32),
            pltpu.VMEM((_CHUNK, 128), jnp.float32),
            pltpu.VMEM((_CHUNK, 128), jnp.float32),
            pltpu.VMEM((feat, b_per_w), jnp.float32),
            pltpu.VMEM((feat, b_per_w), jnp.float32),
            pltpu.SemaphoreType.DMA,
            pltpu.SemaphoreType.DMA,
            pltpu.SemaphoreType.DMA,
            pltpu.SemaphoreType.DMA,
            pltpu.SemaphoreType.DMA,
            pltpu.SemaphoreType.DMA,
            pltpu.SemaphoreType.DMA,
        ],
    )
    def k(table_hbm, idx_hbm, out_hbm, idx_all, pidx_all, g0, g1, g2, g3,
          ot0, ot1, isem, gsem0, gsem1, gsem2, gsem3, osem0, osem1):
        wid = lax.axis_index("s") * _NC + lax.axis_index("c")
        b0 = wid * b_per_w
        iota = lax.iota(jnp.int32, _L)

        gbuf = (g0, g1, g2, g3)
        gsems = (gsem0, gsem1, gsem2, gsem3)
        otile = (ot0, ot1)
        osems = (osem0, osem1)

        icps = [
            pltpu.async_copy(
                idx_hbm.at[pl.ds(f * batch + b0, b_per_w)],
                idx_all.at[pl.ds(f * b_per_w, b_per_w)], isem)
            for f in range(fields)
        ]
        for cp in icps:
            cp.wait()

        def pidx_body(t, carry):
            v = idx_all[pl.ds(t * _L, _L)]
            pidx_all[pl.ds(t * _L, _L)] = lax.shift_right_logical(v, 3)
            return carry
        lax.fori_loop(0, fields * b_per_w // _L, pidx_body, 0)

        def ichunk(u):
            return pidx_all.at[pl.ds(u * _CHUNK, _CHUNK)]

        def fire(u, par):
            pltpu.async_copy(table_hbm.at[ichunk(u)], gbuf[par], gsems[par])

        def gwait(u, par):
            pltpu.make_async_copy(
                table_hbm.at[ichunk(u)], gbuf[par], gsems[par]).wait()

        def extract(u, par, o_ref, col0):
            g = gbuf[par]
            for t in range(_CHUNK // _L):
                rows = iota + t * _L
                v = idx_all[pl.ds(u * _CHUNK + t * _L, _L)]
                off = lax.shift_left(jnp.bitwise_and(v, 7), 4)
                vals = [plsc.load_gather(g, [rows, off + j])
                        for j in range(feat)]
                for j in range(feat):
                    o_ref[j, pl.ds(col0 + t * _L, _L)] = vals[j]

        def oslice(f):
            return out_hbm.at[f, :, pl.ds(b0, b_per_w)]

        for u in range(_NBUF - 1):
            fire(u, u)

        def body(s, carry):
            u0 = s * 2 * nq
            fa = s * 2
            fb = s * 2 + 1

            @pl.when(s > 0)
            def _():
                pltpu.make_async_copy(otile[0], oslice(fa), osems[0]).wait()
                pltpu.make_async_copy(otile[1], oslice(fb), osems[1]).wait()

            for p in range(2 * nq):
                u = u0 + p
                par = p % _NBUF

                @pl.when(u + _NBUF - 1 < nu)
                def _():
                    fire(u + _NBUF - 1, (p + _NBUF - 1) % _NBUF)
                gwait(u, par)
                extract(u, par, otile[p // nq], (p % nq) * _CHUNK)

            pltpu.async_copy(otile[0], oslice(fa), osems[0])
            pltpu.async_copy(otile[1], oslice(fb), osems[1])
            return carry

        lax.fori_loop(0, fields // 2, body, 0)
        pltpu.make_async_copy(otile[0], oslice(fields - 2), osems[0]).wait()
        pltpu.make_async_copy(otile[1], oslice(fields - 1), osems[1]).wait()

    return k


def kernel(inputs, embedding):
    batch, fields = inputs.shape
    num_emb, feat = embedding.shape
    idx_fm = jnp.transpose(inputs).reshape(batch * fields).astype(jnp.int32)
    emb_t = jnp.transpose(embedding)
    n_tail = num_emb - (num_emb // 128) * 128
    tail_packed = jnp.reshape(
        embedding[num_emb - n_tail:], (n_tail // 8, 8 * feat))
    packed = _make_relayout(num_emb, feat)(emb_t, tail_packed)
    call = _make_lookup(batch, fields, feat, num_emb)
    out_t = call(packed, idx_fm)
    return jnp.transpose(out_t, (2, 0, 1))


# speedup vs baseline: 1.7996x; 1.0751x over previous
import jax
import jax.numpy as jnp
from jax import lax
from jax.experimental import pallas as pl
from jax.experimental.pallas import tpu as pltpu
from jax.experimental.pallas import tpu_sc as plsc

_NC = 2
_NS = 16
_NW = _NC * _NS
_L = 16

_CHUNK = 128
_NBUF = 4


def _make_relayout(num_emb, feat):
    n_packed = num_emb // 8
    n_full = num_emb // 128
    tail = num_emb - n_full * 128
    per_w = n_full // _NW
    rem = n_full - per_w * _NW
    nslots = (per_w + (1 if rem else 0) + _NBUF - 1) // _NBUF

    mesh = plsc.VectorSubcoreMesh(
        core_axis_name="c", subcore_axis_name="s",
        num_cores=_NC, num_subcores=_NS)

    @pl.kernel(
        out_type=jax.ShapeDtypeStruct((n_packed, 128), jnp.float32),
        mesh=mesh,
        compiler_params=pltpu.CompilerParams(
            use_tc_tiling_on_sc=True, needs_layout_passes=False,
            disable_bounds_checks=True),
        scratch_types=[
            pltpu.VMEM((feat, 128), jnp.float32),
            pltpu.VMEM((feat, 128), jnp.float32),
            pltpu.VMEM((feat, 128), jnp.float32),
            pltpu.VMEM((feat, 128), jnp.float32),
            pltpu.VMEM((feat, 128), jnp.float32),
            pltpu.VMEM((feat, 128), jnp.float32),
            pltpu.SemaphoreType.DMA,
            pltpu.SemaphoreType.DMA,
            pltpu.SemaphoreType.DMA,
            pltpu.SemaphoreType.DMA,
            pltpu.SemaphoreType.DMA,
            pltpu.SemaphoreType.DMA,
        ],
    )
    def k(emb_t_hbm, tailp_hbm, packed_hbm, a0, a1, a2, a3, o0, o1,
          s0, s1, s2, s3, os0, os1):
        wid = lax.axis_index("s") * _NC + lax.axis_index("c")
        iota = lax.iota(jnp.int32, _L)
        abuf = (a0, a1, a2, a3)
        asem = (s0, s1, s2, s3)
        obuf = (o0, o1)
        osem = (os0, os1)

        def cof(t):
            return wid + t * _NW

        def valid(t):
            return cof(t) < n_full

        def fire(t, par):
            pltpu.async_copy(
                emb_t_hbm.at[:, pl.ds(pl.multiple_of(cof(t) * 128, 128), 128)],
                abuf[par], asem[par])

        def await_in(par):
            pltpu.make_async_copy(
                emb_t_hbm.at[:, pl.ds(0, 128)], abuf[par], asem[par]).wait()

        def transpose(src, dst):
            for ab in range(2):
                vals = [
                    plsc.load_gather(
                        src, [iota, jnp.full((_L,), ab * 64 + g, jnp.int32)])
                    for g in range(48)
                ]
                for g in range(48):
                    l = ab * 64 + g
                    dst[l // 8, pl.ds((l % 8) * _L, _L)] = vals[g]
                vals = [
                    plsc.load_gather(
                        src, [iota, jnp.full((_L,), ab * 64 + 48 + g,
                                             jnp.int32)])
                    for g in range(16)
                ]
                for g in range(16):
                    l = ab * 64 + 48 + g
                    dst[l // 8, pl.ds((l % 8) * _L, _L)] = vals[g]

        def flush(t, opar):
            pltpu.async_copy(
                obuf[opar],
                packed_hbm.at[pl.ds(pl.multiple_of(cof(t) * 16, 16), 16), :],
                osem[opar])

        def drain_out(opar):
            pltpu.make_async_copy(
                obuf[opar], packed_hbm.at[pl.ds(0, 16), :],
                osem[opar]).wait()

        for t in range(_NBUF - 1):
            @pl.when(valid(t))
            def _(t=t):
                fire(t, t)

        def body(s, carry):
            for p in range(_NBUF):
                t = s * _NBUF + p

                @pl.when(valid(t + _NBUF - 1))
                def _():
                    fire(t + _NBUF - 1, (p + _NBUF - 1) % _NBUF)

                @pl.when(valid(t))
                def _():
                    await_in(p)

                    @pl.when(t >= 2)
                    def _():
                        drain_out(p % 2)
                    transpose(abuf[p], obuf[p % 2])
                    flush(t, p % 2)
            return carry

        lax.fori_loop(0, nslots, body, 0)

        drain_out(0)
        drain_out(1)

        if tail:
            @pl.when(wid == 0)
            def _():
                pltpu.sync_copy(tailp_hbm, obuf[0].at[pl.ds(0, tail // 8), :])
                pltpu.sync_copy(
                    obuf[0].at[pl.ds(0, tail // 8), :],
                    packed_hbm.at[pl.ds(n_full * 16, tail // 8), :])

    return k


def _make_lookup(batch, fields, feat, num_emb):
    assert batch % _NW == 0
    b_per_w = batch // _NW
    nq = b_per_w // _CHUNK
    assert nq == _NBUF and fields % 2 == 0
    nu = fields * nq

    mesh = plsc.VectorSubcoreMesh(
        core_axis_name="c", subcore_axis_name="s",
        num_cores=_NC, num_subcores=_NS)

    @pl.kernel(
        out_type=jax.ShapeDtypeStruct((fields, feat, batch), jnp.float32),
        mesh=mesh,
        compiler_params=pltpu.CompilerParams(
            use_tc_tiling_on_sc=True, needs_layout_passes=False),
        scratch_types=[
            pltpu.VMEM((fields * b_per_w,), jnp.int32),
            pltpu.VMEM((fields * b_per_w,), jnp.int32),
            pltpu.VMEM((_CHUNK, 128), jnp.float32),
            pltpu.VMEM((_CHUNK, 128), jnp.float32),
            pltpu.VMEM((_CHUNK, 128), jnp.float32),
            pltpu.VMEM((_CHUNK, 128), jnp.float32),
            pltpu.VMEM((feat, b_per_w), jnp.float32),
            pltpu.VMEM((feat, b_per_w), jnp.float32),
            pltpu.SemaphoreType.DMA,
            pltpu.SemaphoreType.DMA,
            pltpu.SemaphoreType.DMA,
            pltpu.SemaphoreType.DMA,
            pltpu.SemaphoreType.DMA,
            pltpu.SemaphoreType.DMA,
            pltpu.SemaphoreType.DMA,
        ],
    )
    def k(table_hbm, idx_hbm, out_hbm, idx_all, pidx_all, g0, g1, g2, g3,
          ot0, ot1, isem, gsem0, gsem1, gsem2, gsem3, osem0, osem1):
        wid = lax.axis_index("s") * _NC + lax.axis_index("c")
        b0 = pl.multiple_of(wid * b_per_w, b_per_w)
        iota = lax.iota(jnp.int32, _L)

        gbuf = (g0, g1, g2, g3)
        gsems = (gsem0, gsem1, gsem2, gsem3)
        otile = (ot0, ot1)
        osems = (osem0, osem1)

        icps = [
            pltpu.async_copy(
                idx_hbm.at[pl.ds(f * batch + b0, b_per_w)],
                idx_all.at[pl.ds(f * b_per_w, b_per_w)], isem)
            for f in range(fields)
        ]
        for cp in icps:
            cp.wait()

        def pidx_body(t, carry):
            v = idx_all[pl.ds(t * _L, _L)]
            pidx_all[pl.ds(t * _L, _L)] = lax.shift_right_logical(v, 3)
            return carry
        lax.fori_loop(0, fields * b_per_w // _L, pidx_body, 0)

        def ichunk(u):
            return pidx_all.at[pl.ds(u * _CHUNK, _CHUNK)]

        def fire(u, par):
            pltpu.async_copy(table_hbm.at[ichunk(u)], gbuf[par], gsems[par])

        def gwait(u, par):
            pltpu.make_async_copy(
                table_hbm.at[ichunk(u)], gbuf[par], gsems[par]).wait()

        def extract(u, par, o_ref, col0):
            g = gbuf[par]
            for t in range(_CHUNK // _L):
                rows = iota + t * _L
                v = idx_all[pl.ds(u * _CHUNK + t * _L, _L)]
                off = lax.shift_left(jnp.bitwise_and(v, 7), 4)
                vals = [plsc.load_gather(g, [rows, off + j])
                        for j in range(feat)]
                for j in range(feat):
                    o_ref[j, pl.ds(col0 + t * _L, _L)] = vals[j]

        def oslice(f):
            return out_hbm.at[f, :, pl.ds(b0, b_per_w)]

        for u in range(_NBUF - 1):
            fire(u, u)

        def body(s, carry):
            u0 = s * 2 * nq
            fa = s * 2
            fb = s * 2 + 1

            @pl.when(s > 0)
            def _():
                pltpu.make_async_copy(otile[0], oslice(fa), osems[0]).wait()
                pltpu.make_async_copy(otile[1], oslice(fb), osems[1]).wait()

            for p in range(2 * nq):
                u = u0 + p
                par = p % _NBUF

                @pl.when(u + _NBUF - 1 < nu)
                def _():
                    fire(u + _NBUF - 1, (p + _NBUF - 1) % _NBUF)
                gwait(u, par)
                extract(u, par, otile[p // nq], (p % nq) * _CHUNK)

            pltpu.async_copy(otile[0], oslice(fa), osems[0])
            pltpu.async_copy(otile[1], oslice(fb), osems[1])
            return carry

        lax.fori_loop(0, fields // 2, body, 0)
        pltpu.make_async_copy(otile[0], oslice(fields - 2), osems[0]).wait()
        pltpu.make_async_copy(otile[1], oslice(fields - 1), osems[1]).wait()

    return k


def kernel(inputs, embedding):
    batch, fields = inputs.shape
    num_emb, feat = embedding.shape
    idx_fm = jnp.transpose(inputs).reshape(batch * fields).astype(jnp.int32)
    emb_t = jnp.transpose(embedding)
    n_tail = num_emb - (num_emb // 128) * 128
    tail_packed = jnp.reshape(
        embedding[num_emb - n_tail:], (n_tail // 8, 8 * feat))
    packed = _make_relayout(num_emb, feat)(emb_t, tail_packed)
    call = _make_lookup(batch, fields, feat, num_emb)
    out_t = call(packed, idx_fm)
    return jnp.transpose(out_t, (2, 0, 1))
